# Initial kernel scaffold; baseline (speedup 1.0000x reference)
import jax
import jax.numpy as jnp
from jax.experimental import pallas as pl


def kernel(node_types, edge_index, edge_types, graph_ids, target_idx, node_emb, edge_emb, W_msg, W_self, W_g):
    raise NotImplementedError("write your pallas kernel here")


# speedup vs baseline: 8.6101x; 8.6101x over previous
import functools

import jax
import jax.numpy as jnp
from jax import lax
from jax.experimental import pallas as pl
from jax.experimental.pallas import tpu as pltpu
from jax.experimental.pallas import tpu_sc as plsc

N = 10000
E = 160000
D = 256
B = 64
NT = 64
ET = 8

NC = 2
NS = 16
LANES = 16

EP = 163840
PAD = EP - E
EDGES_A = EP // (NC * NS)
ROWS_A = EDGES_A // 128
EDGES_B = EP // NS
ROWS_B = EDGES_B // 128

NPAD = N + 240
C2_SH = NPAD * NT
C_SH = NPAD * ET
C2_T = C2_SH // NS
C_T = C_SH // NS
NROW_T = NPAD // NS

_mesh = plsc.VectorSubcoreMesh(core_axis_name="c", subcore_axis_name="s")


@functools.partial(
    pl.kernel,
    out_type=(
        jax.ShapeDtypeStruct((NC * C2_SH,), jnp.float32),
        jax.ShapeDtypeStruct((NC * C_SH,), jnp.float32),
    ),
    mesh=_mesh,
    scratch_types=[
        pltpu.VMEM((N,), jnp.int32),
        pltpu.VMEM((ROWS_A, 128), jnp.int32),
        pltpu.VMEM((ROWS_A, 128), jnp.int32),
        pltpu.VMEM((ROWS_A, 128), jnp.int32),
        pltpu.VMEM((ROWS_A, 128), jnp.int32),
        pltpu.VMEM((ROWS_A, 128), jnp.int32),
        pltpu.VMEM((128,), jnp.float32),
        pltpu.VMEM_SHARED((C2_SH,), jnp.float32),
        pltpu.VMEM_SHARED((C_SH,), jnp.float32),
    ],
    compiler_params=pltpu.CompilerParams(needs_layout_passes=False),
)
def _counts_kernel(srcp, dstp, etp, nts, zeros_hbm, out_c2, out_c,
                   nt_v, src_v, dst_v, et_v, idx2_v, idxc_v, ones_v,
                   c2_sh, c_sh):
    c = lax.axis_index("c")
    s = lax.axis_index("s")

    pltpu.sync_copy(zeros_hbm.at[pl.ds(s * C2_T, C2_T)],
                    c2_sh.at[pl.ds(s * C2_T, C2_T)])
    pltpu.sync_copy(zeros_hbm.at[pl.ds(s * C_T, C_T)],
                    c_sh.at[pl.ds(s * C_T, C_T)])

    pltpu.sync_copy(nts, nt_v)
    pltpu.sync_copy(srcp.at[c, s], src_v)
    pltpu.sync_copy(dstp.at[c, s], dst_v)
    pltpu.sync_copy(etp.at[c, s], et_v)

    for k in range(8):
        ones_v[pl.ds(k * LANES, LANES)] = jnp.ones((LANES,), jnp.float32)

    def idx_body(r, carry):
        for k in range(8):
            sl = pl.ds(k * LANES, LANES)
            sv = src_v[r, sl]
            nt16 = plsc.load_gather(nt_v, [sv])
            dv = dst_v[r, sl]
            ev = et_v[r, sl]
            idx2_v[r, sl] = dv * NT + nt16
            idxc_v[r, sl] = dv * ET + ev
        return carry

    lax.fori_loop(0, ROWS_A, idx_body, 0)
    plsc.subcore_barrier()

    def add_body(r, carry):
        pltpu.sync_copy(ones_v, c2_sh.at[idx2_v.at[r]], add=True)
        pltpu.sync_copy(ones_v, c_sh.at[idxc_v.at[r]], add=True)
        return carry

    lax.fori_loop(0, ROWS_A, add_body, 0)
    plsc.subcore_barrier()

    pltpu.sync_copy(c2_sh.at[pl.ds(s * C2_T, C2_T)],
                    out_c2.at[pl.ds(c * C2_SH + s * C2_T, C2_T)])
    pltpu.sync_copy(c_sh.at[pl.ds(s * C_T, C_T)],
                    out_c.at[pl.ds(c * C_SH + s * C_T, C_T)])


@functools.partial(
    pl.kernel,
    out_type=jax.ShapeDtypeStruct((NC, NPAD, 128), jnp.float32),
    mesh=_mesh,
    scratch_types=[
        pltpu.VMEM((ROWS_B, 128), jnp.int32),
        pltpu.VMEM((ROWS_B, 128), jnp.int32),
        pltpu.VMEM((128, 128), jnp.float32),
        pltpu.VMEM_SHARED((NPAD, 128), jnp.float32),
        pltpu.SemaphoreType.DMA,
    ],
)
def _spmm_kernel(hw_flat, srcb, dstb, zeros2d, out,
                 src_v, dst_v, rows_v, agg_sh, gsem):
    c = lax.axis_index("c")
    s = lax.axis_index("s")

    pltpu.sync_copy(zeros2d.at[pl.ds(s * NROW_T, NROW_T)],
                    agg_sh.at[pl.ds(s * NROW_T, NROW_T)])
    pltpu.sync_copy(srcb.at[c, s], src_v)
    pltpu.sync_copy(dstb.at[s], dst_v)
    plsc.subcore_barrier()

    def body(j, carry):
        pltpu.async_copy(hw_flat.at[src_v.at[j]], rows_v, gsem).wait()
        pltpu.sync_copy(rows_v, agg_sh.at[dst_v.at[j]], add=True)
        return carry

    lax.fori_loop(0, ROWS_B, body, 0)
    plsc.subcore_barrier()

    pltpu.sync_copy(agg_sh.at[pl.ds(s * NROW_T, NROW_T)],
                    out.at[c, pl.ds(s * NROW_T, NROW_T)])


def _layer1_body(nts_ref, c2_ref, c_ref, nemb_ref, eemb_ref, wm_ref, ws_ref,
                 hw_ref, z_ref):
    f32 = jnp.float32
    t0 = jnp.dot(nemb_ref[...], wm_ref[0], preferred_element_type=f32)
    tw0 = jnp.dot(eemb_ref[...], wm_ref[0], preferred_element_type=f32)
    tw1 = jnp.dot(eemb_ref[...], wm_ref[1], preferred_element_type=f32)
    s0 = jnp.dot(nemb_ref[...], ws_ref[0], preferred_element_type=f32)

    nt = nts_ref[0, 0, :]
    oh = (nt[:, None] == lax.broadcasted_iota(jnp.int32, (nt.shape[0], NT), 1)
          ).astype(f32)
    acc = jnp.dot(oh, s0, preferred_element_type=f32)
    acc += jnp.dot(c2_ref[...], jnp.concatenate([t0, t0], 0),
                   preferred_element_type=f32)
    acc += jnp.dot(c_ref[...], jnp.concatenate([tw0, tw0], 0),
                   preferred_element_type=f32)
    h1 = jnp.maximum(acc, 0.0)
    hw_ref[...] = jnp.dot(h1, wm_ref[1], preferred_element_type=f32)
    z_ref[...] = (jnp.dot(h1, ws_ref[1], preferred_element_type=f32)
                  + jnp.dot(c_ref[...], jnp.concatenate([tw1, tw1], 0),
                            preferred_element_type=f32))


def _layer1(nts_r, c2cat, ccat, node_emb, edge_emb, w_msg, w_self):
    nb = 10
    blk = N // nb
    return pl.pallas_call(
        _layer1_body,
        grid=(nb,),
        in_specs=[
            pl.BlockSpec((1, 1, blk), lambda i: (i, 0, 0)),
            pl.BlockSpec((blk, 2 * NT), lambda i: (i, 0)),
            pl.BlockSpec((blk, 2 * ET), lambda i: (i, 0)),
            pl.BlockSpec((NT, D), lambda i: (0, 0)),
            pl.BlockSpec((ET, D), lambda i: (0, 0)),
            pl.BlockSpec((2, D, D), lambda i: (0, 0, 0)),
            pl.BlockSpec((2, D, D), lambda i: (0, 0, 0)),
        ],
        out_specs=[
            pl.BlockSpec((blk, D), lambda i: (i, 0)),
            pl.BlockSpec((blk, D), lambda i: (i, 0)),
        ],
        out_shape=[
            jax.ShapeDtypeStruct((N, D), jnp.float32),
            jax.ShapeDtypeStruct((N, D), jnp.float32),
        ],
    )(nts_r, c2cat, ccat, node_emb, edge_emb, w_msg, w_self)


def _readout_body(z_ref, agg_ref, gid_ref, tgt_ref, pooled_ref, cnt_ref,
                  tgte_ref):
    f32 = jnp.float32
    i = pl.program_id(0)
    blk = z_ref.shape[0]
    h2 = jnp.maximum(z_ref[...] + agg_ref[...], 0.0)

    gids = gid_ref[0, 0, :]
    g_oh = (gids[None, :] == lax.broadcasted_iota(jnp.int32, (B, blk), 0)
            ).astype(f32)
    rows = i * blk + lax.broadcasted_iota(jnp.int32, (B, blk), 1)
    t_oh = (tgt_ref[0, :][:, None] == rows).astype(f32)

    pooled_c = jnp.dot(g_oh, h2, preferred_element_type=f32)
    cnt_c = jnp.broadcast_to(jnp.sum(g_oh, axis=1)[:, None], (B, D))
    tgt_c = jnp.dot(t_oh, h2, preferred_element_type=f32)

    @pl.when(i == 0)
    def _():
        pooled_ref[...] = pooled_c
        cnt_ref[...] = cnt_c
        tgte_ref[...] = tgt_c

    @pl.when(i > 0)
    def _():
        pooled_ref[...] += pooled_c
        cnt_ref[...] += cnt_c
        tgte_ref[...] += tgt_c


def _readout(z1, aggcat, gid_r, tgt_r):
    nb = 10
    blk = N // nb
    return pl.pallas_call(
        _readout_body,
        grid=(nb,),
        in_specs=[
            pl.BlockSpec((blk, D), lambda i: (i, 0)),
            pl.BlockSpec((blk, D), lambda i: (i, 0)),
            pl.BlockSpec((1, 1, blk), lambda i: (i, 0, 0)),
            pl.BlockSpec((1, B), lambda i: (0, 0)),
        ],
        out_specs=[
            pl.BlockSpec((B, D), lambda i: (0, 0)),
            pl.BlockSpec((B, D), lambda i: (0, 0)),
            pl.BlockSpec((B, D), lambda i: (0, 0)),
        ],
        out_shape=[
            jax.ShapeDtypeStruct((B, D), jnp.float32),
            jax.ShapeDtypeStruct((B, D), jnp.float32),
            jax.ShapeDtypeStruct((B, D), jnp.float32),
        ],
    )(z1, aggcat, gid_r, tgt_r)


def _head_body(pooled_ref, cnt_ref, tgte_ref, wg_ref, out_ref):
    ge = jnp.tanh(jnp.dot(pooled_ref[...] / jnp.maximum(cnt_ref[...], 1.0),
                          wg_ref[...], preferred_element_type=jnp.float32))
    out_ref[...] = jnp.concatenate([ge, tgte_ref[...]], axis=1)


def _head(pooled, cnt, tgte, w_g):
    return pl.pallas_call(
        _head_body,
        out_shape=jax.ShapeDtypeStruct((B, 2 * D), jnp.float32),
    )(pooled, cnt, tgte, w_g)


def kernel(node_types, edge_index, edge_types, graph_ids, target_idx,
           node_emb, edge_emb, W_msg, W_self, W_g):
    i32 = jnp.int32
    src = edge_index[0].astype(i32)
    dst = edge_index[1].astype(i32)
    et = edge_types.astype(i32)

    pad_ar = jnp.arange(PAD, dtype=i32)
    src_p = jnp.concatenate([src, pad_ar % N])
    dst_p = jnp.concatenate([dst, N + (pad_ar % 64)])
    et_p = jnp.concatenate([et, jnp.zeros((PAD,), i32)])

    srcp_a = src_p.reshape(NC, NS, ROWS_A, 128)
    dstp_a = dst_p.reshape(NC, NS, ROWS_A, 128)
    etp_a = et_p.reshape(NC, NS, ROWS_A, 128)

    srcb = jnp.stack([src_p, src_p + N]).reshape(NC, NS, ROWS_B, 128)
    dstb = dst_p.reshape(NS, ROWS_B, 128)

    zeros1d = jnp.zeros((C2_SH,), jnp.float32)
    zeros2d = jnp.zeros((NPAD, 128), jnp.float32)

    nts = node_types.astype(i32)
    c2p, cp = _counts_kernel(srcp_a, dstp_a, etp_a, nts, zeros1d)
    c2p = c2p.reshape(NC, C2_SH)
    cp = cp.reshape(NC, C_SH)
    c2cat = jnp.concatenate(
        [c2p[0, :N * NT].reshape(N, NT), c2p[1, :N * NT].reshape(N, NT)],
        axis=1)
    ccat = jnp.concatenate(
        [cp[0, :N * ET].reshape(N, ET), cp[1, :N * ET].reshape(N, ET)],
        axis=1)

    nts_r = nts.reshape(10, 1, N // 10)
    hw1, z1 = _layer1(nts_r, c2cat, ccat, node_emb, edge_emb, W_msg, W_self)

    hw_flat = jnp.concatenate([hw1[:, :128], hw1[:, 128:]], axis=0)
    aggp = _spmm_kernel(hw_flat, srcb, dstb, zeros2d)
    aggcat = jnp.concatenate([aggp[0, :N], aggp[1, :N]], axis=1)

    gid_r = graph_ids.astype(i32).reshape(10, 1, N // 10)
    tgt_r = target_idx.astype(i32).reshape(1, B)
    pooled, cnt, tgte = _readout(z1, aggcat, gid_r, tgt_r)

    return _head(pooled, cnt, tgte, W_g)


# speedup vs baseline: 13.9789x; 1.6235x over previous
import functools

import jax
import jax.numpy as jnp
from jax import lax
from jax.experimental import pallas as pl
from jax.experimental.pallas import tpu as pltpu
from jax.experimental.pallas import tpu_sc as plsc

N = 10000
E = 160000
D = 256
B = 64
NT = 64
ET = 8

NC = 2
NS = 16
LANES = 16

EP = 163840
PAD = EP - E
EDGES_A = EP // (NC * NS)
ROWS_A = EDGES_A // 128
EDGES_B = EP // NS
ROWS_B = EDGES_B // 128

NPAD = N + 240
C2_SH = NPAD * NT
C_SH = NPAD * ET
C2_REAL = N * NT
C_REAL = N * ET
C2_T = 40960
C2_LAST = C2_REAL - C2_T
C_T = 5120
C_LAST = C_REAL - C_T
NROW_T = 640
NROW_LAST = N - NROW_T

_mesh = plsc.VectorSubcoreMesh(core_axis_name="c", subcore_axis_name="s")


@functools.partial(
    pl.kernel,
    out_type=(
        jax.ShapeDtypeStruct((NC * C2_REAL,), jnp.float32),
        jax.ShapeDtypeStruct((NC * C_REAL,), jnp.float32),
    ),
    mesh=_mesh,
    scratch_types=[
        pltpu.VMEM((N,), jnp.int32),
        pltpu.VMEM((ROWS_A, 128), jnp.int32),
        pltpu.VMEM((ROWS_A, 128), jnp.int32),
        pltpu.VMEM((ROWS_A, 128), jnp.int32),
        pltpu.VMEM((ROWS_A, 128), jnp.int32),
        pltpu.VMEM((ROWS_A, 128), jnp.int32),
        pltpu.VMEM((128,), jnp.float32),
        pltpu.VMEM_SHARED((C2_SH,), jnp.float32),
        pltpu.VMEM_SHARED((C_SH,), jnp.float32),
    ],
    compiler_params=pltpu.CompilerParams(needs_layout_passes=False),
)
def _counts_kernel(srcp, dstp, etp, nts, zeros_hbm, out_c2, out_c,
                   nt_v, src_v, dst_v, et_v, idx2_v, idxc_v, ones_v,
                   c2_sh, c_sh):
    c = lax.axis_index("c")
    s = lax.axis_index("s")
    c2_off = pl.multiple_of(jnp.minimum(s * C2_T, C2_LAST), 128)
    c_off = pl.multiple_of(jnp.minimum(s * C_T, C_LAST), 128)

    pltpu.sync_copy(zeros_hbm.at[pl.ds(c2_off, C2_T)],
                    c2_sh.at[pl.ds(c2_off, C2_T)])
    pltpu.sync_copy(zeros_hbm.at[pl.ds(c_off, C_T)],
                    c_sh.at[pl.ds(c_off, C_T)])

    pltpu.sync_copy(nts, nt_v)
    pltpu.sync_copy(srcp.at[c, s], src_v)
    pltpu.sync_copy(dstp.at[c, s], dst_v)
    pltpu.sync_copy(etp.at[c, s], et_v)

    for k in range(8):
        ones_v[pl.ds(k * LANES, LANES)] = jnp.ones((LANES,), jnp.float32)

    def idx_body(r, carry):
        for k in range(8):
            sl = pl.ds(k * LANES, LANES)
            sv = src_v[r, sl]
            nt16 = plsc.load_gather(nt_v, [sv])
            dv = dst_v[r, sl]
            ev = et_v[r, sl]
            idx2_v[r, sl] = dv * NT + nt16
            idxc_v[r, sl] = dv * ET + ev
        return carry

    lax.fori_loop(0, ROWS_A, idx_body, 0)
    plsc.subcore_barrier()

    def add_body(r, carry):
        pltpu.sync_copy(ones_v, c2_sh.at[idx2_v.at[r]], add=True)
        pltpu.sync_copy(ones_v, c_sh.at[idxc_v.at[r]], add=True)
        return carry

    lax.fori_loop(0, ROWS_A, add_body, 0)
    plsc.subcore_barrier()

    pltpu.sync_copy(
        c2_sh.at[pl.ds(c2_off, C2_T)],
        out_c2.at[pl.ds(pl.multiple_of(c * C2_REAL + c2_off, 128), C2_T)])
    pltpu.sync_copy(
        c_sh.at[pl.ds(c_off, C_T)],
        out_c.at[pl.ds(pl.multiple_of(c * C_REAL + c_off, 128), C_T)])


HROWS = ROWS_B // 2
@functools.partial(
    pl.kernel,
    out_type=jax.ShapeDtypeStruct((NC, N, 128), jnp.float32),
    mesh=_mesh,
    scratch_types=[
        pltpu.VMEM((HROWS, 128), jnp.int32),
        pltpu.VMEM((ROWS_B, 128), jnp.int32),
        pltpu.VMEM((2, 128, 128), jnp.float32),
        pltpu.VMEM_SHARED((N + 64, 128), jnp.float32),
        pltpu.SemaphoreType.DMA,
    ],
)
def _spmm_kernel(hw_flat, srcb, dstb, zeros2d, out,
                 src_v, dst_v, rows_v, agg_sh, gsem):
    c = lax.axis_index("c")
    s = lax.axis_index("s")
    r_off = pl.multiple_of(jnp.minimum(s * NROW_T, NROW_LAST), 8)

    pltpu.sync_copy(zeros2d.at[pl.ds(r_off, NROW_T)],
                    agg_sh.at[pl.ds(r_off, NROW_T)])
    pltpu.sync_copy(dstb.at[s], dst_v)
    plsc.subcore_barrier()

    for h in range(2):
        pltpu.sync_copy(srcb.at[c, s, h], src_v)
        pltpu.async_copy(hw_flat.at[src_v.at[0]], rows_v.at[0], gsem)

        def body(j, carry):
            nxt = j + 1

            @pl.when(nxt < HROWS)
            def _():
                pltpu.async_copy(hw_flat.at[src_v.at[nxt]],
                                 rows_v.at[lax.rem(nxt, 2)], gsem)

            b = lax.rem(j, 2)
            pltpu.make_async_copy(hw_flat.at[src_v.at[j]],
                                  rows_v.at[b], gsem).wait()
            pltpu.sync_copy(rows_v.at[b],
                            agg_sh.at[dst_v.at[h * HROWS + j]], add=True)
            return carry

        lax.fori_loop(0, HROWS, body, 0)

    plsc.subcore_barrier()
    pltpu.sync_copy(agg_sh.at[pl.ds(r_off, NROW_T)],
                    out.at[c, pl.ds(r_off, NROW_T)])


def _layer1_body(nts_ref, c2a_ref, c2b_ref, ca_ref, cb_ref, nemb_ref,
                 eemb_ref, wm_ref, ws_ref, hw_ref, z_ref):
    f32 = jnp.float32
    t0 = jnp.dot(nemb_ref[...], wm_ref[0], preferred_element_type=f32)
    tw0 = jnp.dot(eemb_ref[...], wm_ref[0], preferred_element_type=f32)
    tw1 = jnp.dot(eemb_ref[...], wm_ref[1], preferred_element_type=f32)
    s0 = jnp.dot(nemb_ref[...], ws_ref[0], preferred_element_type=f32)

    c2 = c2a_ref[0] + c2b_ref[0]
    cc = ca_ref[0] + cb_ref[0]
    nt = nts_ref[0, 0, :]
    oh = (nt[:, None] == lax.broadcasted_iota(jnp.int32, (nt.shape[0], NT), 1)
          ).astype(f32)
    acc = jnp.dot(oh, s0, preferred_element_type=f32)
    acc += jnp.dot(c2, t0, preferred_element_type=f32)
    acc += jnp.dot(cc, tw0, preferred_element_type=f32)
    h1 = jnp.maximum(acc, 0.0)
    hw = jnp.dot(h1, wm_ref[1], preferred_element_type=f32)
    hw_ref[0] = hw[:, :128]
    hw_ref[1] = hw[:, 128:]
    z_ref[...] = (jnp.dot(h1, ws_ref[1], preferred_element_type=f32)
                  + jnp.dot(cc, tw1, preferred_element_type=f32))


def _layer1(nts_r, c2p, cp, node_emb, edge_emb, w_msg, w_self):
    nb = 10
    blk = N // nb
    return pl.pallas_call(
        _layer1_body,
        grid=(nb,),
        in_specs=[
            pl.BlockSpec((1, 1, blk), lambda i: (i, 0, 0)),
            pl.BlockSpec((1, blk, NT), lambda i: (0, i, 0)),
            pl.BlockSpec((1, blk, NT), lambda i: (1, i, 0)),
            pl.BlockSpec((1, blk, ET), lambda i: (0, i, 0)),
            pl.BlockSpec((1, blk, ET), lambda i: (1, i, 0)),
            pl.BlockSpec((NT, D), lambda i: (0, 0)),
            pl.BlockSpec((ET, D), lambda i: (0, 0)),
            pl.BlockSpec((2, D, D), lambda i: (0, 0, 0)),
            pl.BlockSpec((2, D, D), lambda i: (0, 0, 0)),
        ],
        out_specs=[
            pl.BlockSpec((2, blk, 128), lambda i: (0, i, 0)),
            pl.BlockSpec((blk, D), lambda i: (i, 0)),
        ],
        out_shape=[
            jax.ShapeDtypeStruct((2, N, 128), jnp.float32),
            jax.ShapeDtypeStruct((N, D), jnp.float32),
        ],
    )(nts_r, c2p, c2p, cp, cp, node_emb, edge_emb, w_msg, w_self)


def _readout_body(z_ref, agg0_ref, agg1_ref, gid_ref,
                  tgt_ref, pooled_ref, cnt_ref, tgte_ref):
    f32 = jnp.float32
    i = pl.program_id(0)
    blk = z_ref.shape[0]
    agg = jnp.concatenate([agg0_ref[0], agg1_ref[0]], axis=1)
    h2 = jnp.maximum(z_ref[...] + agg, 0.0)

    gids = gid_ref[0, 0, :]
    g_oh = (gids[None, :] == lax.broadcasted_iota(jnp.int32, (B, blk), 0)
            ).astype(f32)
    rows = i * blk + lax.broadcasted_iota(jnp.int32, (B, blk), 1)
    t_oh = (tgt_ref[0, :][:, None] == rows).astype(f32)

    pooled_c = jnp.dot(g_oh, h2, preferred_element_type=f32)
    cnt_c = jnp.broadcast_to(jnp.sum(g_oh, axis=1)[:, None], (B, D))
    tgt_c = jnp.dot(t_oh, h2, preferred_element_type=f32)

    @pl.when(i == 0)
    def _():
        pooled_ref[...] = pooled_c
        cnt_ref[...] = cnt_c
        tgte_ref[...] = tgt_c

    @pl.when(i > 0)
    def _():
        pooled_ref[...] += pooled_c
        cnt_ref[...] += cnt_c
        tgte_ref[...] += tgt_c


def _readout(z1, aggp, gid_r, tgt_r):
    nb = 10
    blk = N // nb
    return pl.pallas_call(
        _readout_body,
        grid=(nb,),
        in_specs=[
            pl.BlockSpec((blk, D), lambda i: (i, 0)),
            pl.BlockSpec((1, blk, 128), lambda i: (0, i, 0)),
            pl.BlockSpec((1, blk, 128), lambda i: (1, i, 0)),
            pl.BlockSpec((1, 1, blk), lambda i: (i, 0, 0)),
            pl.BlockSpec((1, B), lambda i: (0, 0)),
        ],
        out_specs=[
            pl.BlockSpec((B, D), lambda i: (0, 0)),
            pl.BlockSpec((B, D), lambda i: (0, 0)),
            pl.BlockSpec((B, D), lambda i: (0, 0)),
        ],
        out_shape=[
            jax.ShapeDtypeStruct((B, D), jnp.float32),
            jax.ShapeDtypeStruct((B, D), jnp.float32),
            jax.ShapeDtypeStruct((B, D), jnp.float32),
        ],
    )(z1, aggp, aggp, gid_r, tgt_r)


def _head_body(pooled_ref, cnt_ref, tgte_ref, wg_ref, out_ref):
    ge = jnp.tanh(jnp.dot(pooled_ref[...] / jnp.maximum(cnt_ref[...], 1.0),
                          wg_ref[...], preferred_element_type=jnp.float32))
    out_ref[...] = jnp.concatenate([ge, tgte_ref[...]], axis=1)


def _head(pooled, cnt, tgte, w_g):
    return pl.pallas_call(
        _head_body,
        out_shape=jax.ShapeDtypeStruct((B, 2 * D), jnp.float32),
    )(pooled, cnt, tgte, w_g)


def kernel(node_types, edge_index, edge_types, graph_ids, target_idx,
           node_emb, edge_emb, W_msg, W_self, W_g):
    i32 = jnp.int32
    src = edge_index[0].astype(i32)
    dst = edge_index[1].astype(i32)
    et = edge_types.astype(i32)

    pad_ar = jnp.arange(PAD, dtype=i32)
    src_p = jnp.concatenate([src, pad_ar % N])
    dst_p = jnp.concatenate([dst, N + (pad_ar % 64)])
    et_p = jnp.concatenate([et, jnp.zeros((PAD,), i32)])

    srcp_a = src_p.reshape(NC, NS, ROWS_A, 128)
    dstp_a = dst_p.reshape(NC, NS, ROWS_A, 128)
    etp_a = et_p.reshape(NC, NS, ROWS_A, 128)

    srcb = jnp.stack([src_p, src_p + N]).reshape(NC, NS, 2, HROWS, 128)
    dstb = dst_p.reshape(NS, ROWS_B, 128)

    zeros1d = jnp.zeros((C2_REAL,), jnp.float32)
    zeros2d = jnp.zeros((N, 128), jnp.float32)

    nts = node_types.astype(i32)
    c2p, cp = _counts_kernel(srcp_a, dstp_a, etp_a, nts, zeros1d)
    c2p = c2p.reshape(NC, N, NT)
    cp = cp.reshape(NC, N, ET)

    nts_r = nts.reshape(10, 1, N // 10)
    hw_pair, z1 = _layer1(nts_r, c2p, cp, node_emb, edge_emb, W_msg, W_self)

    hw_flat = hw_pair.reshape(NC * N, 128)
    aggp = _spmm_kernel(hw_flat, srcb, dstb, zeros2d)

    gid_r = graph_ids.astype(i32).reshape(10, 1, N // 10)
    tgt_r = target_idx.astype(i32).reshape(1, B)
    pooled, cnt, tgte = _readout(z1, aggp, gid_r, tgt_r)

    return _head(pooled, cnt, tgte, W_g)
